# baseline (device time: 22612 ns/iter reference)
import jax
import jax.numpy as jnp
from jax import lax
from jax.experimental import pallas as pl
from jax.experimental.pallas import tpu as pltpu

N_DEV = 8
MESH = pl.DeviceIdType.MESH
MASKS = [1, 2, 4, 3, 5, 6, 7]


def _partner(my, mask):
    b = lax.bitwise_xor(my, lax.bitwise_and(lax.shift_right_logical(my, 1), 1))
    pb = lax.bitwise_xor(b, mask)
    return lax.bitwise_xor(pb, lax.bitwise_and(lax.shift_right_logical(pb, 1), 1))


def kernel(x, w_mat):
    m, k_per = x.shape
    _, n = w_mat.shape
    m_per = m // N_DEV

    def body(x_ref, w_ref, out_ref, stage_ref, xa_ref, send_sems, recv_sems):
        my = lax.axis_index("i")

        barrier_sem = pltpu.get_barrier_semaphore()
        for d in range(1, N_DEV):
            pl.semaphore_signal(
                barrier_sem, inc=1,
                device_id=(lax.rem(my + d, N_DEV),),
                device_id_type=MESH,
            )
        pl.semaphore_wait(barrier_sem, N_DEV - 1)

        for t in range(N_DEV):
            stage_ref[t] = x_ref[pl.ds(t * m_per, m_per), :].astype(jnp.bfloat16)

        sends = []
        for mask in MASKS:
            p = _partner(my, mask)
            rdma = pltpu.make_async_remote_copy(
                src_ref=stage_ref.at[pl.ds(p, 1)],
                dst_ref=xa_ref.at[pl.ds(my, 1)],
                send_sem=send_sems.at[mask],
                recv_sem=recv_sems.at[my],
                device_id=(p,),
                device_id_type=MESH,
            )
            rdma.start()
            sends.append(rdma)

        acc = lax.dot_general(
            stage_ref[pl.ds(my, 1)][0],
            w_ref[pl.ds(my * m_per, m_per), :],
            (((1,), (0,)), ((), ())),
            preferred_element_type=jnp.float32,
        )
        for mask in MASKS:
            p = _partner(my, mask)
            pltpu.make_async_remote_copy(
                src_ref=stage_ref.at[pl.ds(0, 1)],
                dst_ref=xa_ref.at[pl.ds(p, 1)],
                send_sem=send_sems.at[0],
                recv_sem=recv_sems.at[p],
                device_id=(p,),
                device_id_type=MESH,
            ).wait_recv()
            acc = acc + lax.dot_general(
                xa_ref[pl.ds(p, 1)][0],
                w_ref[pl.ds(p * m_per, m_per), :],
                (((1,), (0,)), ((), ())),
                preferred_element_type=jnp.float32,
            )

        out_ref[:, :] = jnp.maximum(acc, 0.0)

        for s in sends:
            s.wait_send()

    return pl.pallas_call(
        body,
        out_shape=jax.ShapeDtypeStruct((m_per, n), jnp.float32),
        in_specs=[
            pl.BlockSpec(memory_space=pltpu.VMEM),
            pl.BlockSpec(memory_space=pltpu.VMEM),
        ],
        out_specs=pl.BlockSpec(memory_space=pltpu.VMEM),
        scratch_shapes=[
            pltpu.VMEM((N_DEV, m_per, k_per), jnp.bfloat16),
            pltpu.VMEM((N_DEV, m_per, k_per), jnp.bfloat16),
            pltpu.SemaphoreType.DMA((N_DEV,)),
            pltpu.SemaphoreType.DMA((N_DEV,)),
        ],
        compiler_params=pltpu.CompilerParams(collective_id=5),
    )(x, w_mat)
